# baseline (speedup 1.0000x reference)
import jax
import jax.numpy as jnp
from jax.experimental import pallas as pl
from jax.experimental.pallas import tpu as pltpu

_B = 16
_T = 64
_L = 64
_H = 768
_E = 63
_C = 4
_TC = _T // _C
_LC = _TC * _L


def _gcn_pool_kernel(hs_ref, msk_ref, edge_ref, wf_ref, bf_ref,
                     w1_ref, b1_ref, w2_ref, b2_ref, wfc_ref, bfc_ref,
                     out_ref, nodes_ref):
    f32 = jnp.float32
    c = pl.program_id(1)

    hs = hs_ref[0]
    m = msk_ref[0]
    col_tweet = jax.lax.broadcasted_iota(jnp.int32, (_TC, _LC), 1) // _L
    row_tweet = jax.lax.broadcasted_iota(jnp.int32, (_TC, _LC), 0)
    pool = jnp.where(col_tweet == row_tweet,
                     jnp.broadcast_to(m, (_TC, _LC)), 0.0)
    sums = jnp.dot(pool, hs, preferred_element_type=f32)
    cnts = jnp.sum(pool, axis=1, keepdims=True)
    nodes_ref[pl.ds(c * _TC, _TC), :] = sums / jnp.maximum(cnts, 1e-9)

    @pl.when(c == _C - 1)
    def _graph():
        nodes = nodes_ref[...]

        src = edge_ref[0, 0:1, :]
        dst = edge_ref[0, 1:2, :]
        node_ids = jax.lax.broadcasted_iota(jnp.int32, (_T, _E), 0)
        oh_srcT = (node_ids == src).astype(f32)
        oh_dstT = (node_ids == dst).astype(f32)
        s_nodes = jax.lax.dot_general(wf_ref[...], nodes,
                                      (((1,), (1,)), ((), ())),
                                      preferred_element_type=f32)
        sc_e = jnp.dot(s_nodes[0:1, :], oh_dstT, preferred_element_type=f32)
        sp_e = jnp.dot(s_nodes[1:2, :], oh_srcT, preferred_element_type=f32)
        ew = jax.nn.sigmoid(sc_e + sp_e + bf_ref[0, 0])

        a_w = jax.lax.dot_general(oh_dstT * ew, oh_srcT,
                                  (((1,), (1,)), ((), ())),
                                  preferred_element_type=f32)
        eye = (jax.lax.broadcasted_iota(jnp.int32, (_T, _T), 0)
               == jax.lax.broadcasted_iota(jnp.int32, (_T, _T), 1)).astype(f32)
        a_sl = a_w + eye
        deg = jnp.sum(a_sl, axis=1, keepdims=True)
        dinv = jnp.where(deg > 0,
                         jax.lax.rsqrt(jnp.maximum(deg, 1e-12)), 0.0)

        h1 = jnp.dot(nodes, w1_ref[...], preferred_element_type=f32)
        x1 = jnp.maximum(dinv * jnp.dot(a_sl, dinv * h1,
                                        preferred_element_type=f32)
                         + b1_ref[...], 0.0)
        h2 = jnp.dot(x1, w2_ref[...], preferred_element_type=f32)
        x2 = jnp.maximum(dinv * jnp.dot(a_sl, dinv * h2,
                                        preferred_element_type=f32)
                         + b2_ref[...], 0.0)

        pooled = jnp.sum(x2, axis=0, keepdims=True) * (1.0 / _T)
        out_ref[0] = jnp.tanh(jnp.dot(pooled, wfc_ref[...],
                                      preferred_element_type=f32)
                              + bfc_ref[...])


def kernel(hidden_states, attention_msk, tree_lens, edge_index,
           Wf, bf, W1, b1, W2, b2, Wfc, bfc):
    del tree_lens
    msk3 = attention_msk.reshape(_B, 1, _T * _L)
    wf2 = Wf.reshape(2, _H)
    bf2 = bf.reshape(1, 1)
    b1r = b1.reshape(1, _H)
    b2r = b2.reshape(1, _H)
    bfcr = bfc.reshape(1, _H)

    return pl.pallas_call(
        _gcn_pool_kernel,
        grid=(_B, _C),
        in_specs=[
            pl.BlockSpec((1, _LC, _H), lambda i, c: (i, c, 0)),
            pl.BlockSpec((1, 1, _LC), lambda i, c: (i, 0, c)),
            pl.BlockSpec((1, 2, _E), lambda i, c: (i, 0, 0)),
            pl.BlockSpec((2, _H), lambda i, c: (0, 0)),
            pl.BlockSpec((1, 1), lambda i, c: (0, 0)),
            pl.BlockSpec((_H, _H), lambda i, c: (0, 0)),
            pl.BlockSpec((1, _H), lambda i, c: (0, 0)),
            pl.BlockSpec((_H, _H), lambda i, c: (0, 0)),
            pl.BlockSpec((1, _H), lambda i, c: (0, 0)),
            pl.BlockSpec((_H, _H), lambda i, c: (0, 0)),
            pl.BlockSpec((1, _H), lambda i, c: (0, 0)),
        ],
        out_specs=pl.BlockSpec((1, 1, _H), lambda i, c: (i, 0, 0)),
        out_shape=jax.ShapeDtypeStruct((_B, 1, _H), jnp.float32),
        scratch_shapes=[pltpu.VMEM((_T, _H), jnp.float32)],
        compiler_params=pltpu.CompilerParams(
            dimension_semantics=("arbitrary", "arbitrary")),
    )(hidden_states, msk3, edge_index, wf2, bf2,
      W1, b1r, W2, b2r, Wfc, bfcr).reshape(_B, _H)


# speedup vs baseline: 1.4651x; 1.4651x over previous
import jax
import jax.numpy as jnp
from jax.experimental import pallas as pl
from jax.experimental.pallas import tpu as pltpu

_B = 16
_T = 64
_L = 64
_H = 768
_E = 63


def _gcn_pool_kernel(hs_ref, msk_ref, edge_ref, wf_ref, bf_ref,
                     w1_ref, b1_ref, w2_ref, b2_ref, wfc_ref, bfc_ref,
                     out_ref):
    f32 = jnp.float32
    hs = hs_ref[0]
    m = msk_ref[0]

    col_tweet = jax.lax.broadcasted_iota(jnp.int32, (_T, _T * _L), 1) // _L
    row_tweet = jax.lax.broadcasted_iota(jnp.int32, (_T, _T * _L), 0)
    pool = jnp.where(col_tweet == row_tweet,
                     jnp.broadcast_to(m, (_T, _T * _L)), 0.0)
    sums = jnp.dot(pool, hs, preferred_element_type=f32)
    cnts = jnp.sum(pool, axis=1, keepdims=True)
    nodes = sums / jnp.maximum(cnts, 1e-9)

    src = edge_ref[0, 0:1, :]
    dst = edge_ref[0, 1:2, :]
    node_ids = jax.lax.broadcasted_iota(jnp.int32, (_T, _E), 0)
    oh_srcT = (node_ids == src).astype(f32)
    oh_dstT = (node_ids == dst).astype(f32)
    s_nodes = jax.lax.dot_general(wf_ref[...], nodes,
                                  (((1,), (1,)), ((), ())),
                                  preferred_element_type=f32)
    sc_e = jnp.dot(s_nodes[0:1, :], oh_dstT, preferred_element_type=f32)
    sp_e = jnp.dot(s_nodes[1:2, :], oh_srcT, preferred_element_type=f32)
    ew = jax.nn.sigmoid(sc_e + sp_e + bf_ref[0, 0])

    a_w = jax.lax.dot_general(oh_dstT * ew, oh_srcT,
                              (((1,), (1,)), ((), ())),
                              preferred_element_type=f32)
    eye = (jax.lax.broadcasted_iota(jnp.int32, (_T, _T), 0)
           == jax.lax.broadcasted_iota(jnp.int32, (_T, _T), 1)).astype(f32)
    a_sl = a_w + eye
    deg = jnp.sum(a_sl, axis=1, keepdims=True)
    dinv = jnp.where(deg > 0,
                     jax.lax.rsqrt(jnp.maximum(deg, 1e-12)), 0.0)

    h1 = jnp.dot(nodes, w1_ref[...], preferred_element_type=f32)
    x1 = jnp.maximum(dinv * jnp.dot(a_sl, dinv * h1,
                                    preferred_element_type=f32)
                     + b1_ref[...], 0.0)
    h2 = jnp.dot(x1, w2_ref[...], preferred_element_type=f32)
    x2 = jnp.maximum(dinv * jnp.dot(a_sl, dinv * h2,
                                    preferred_element_type=f32)
                     + b2_ref[...], 0.0)

    pooled = jnp.sum(x2, axis=0, keepdims=True) * (1.0 / _T)
    out_ref[0] = jnp.tanh(jnp.dot(pooled, wfc_ref[...],
                                  preferred_element_type=f32)
                          + bfc_ref[...])


def kernel(hidden_states, attention_msk, tree_lens, edge_index,
           Wf, bf, W1, b1, W2, b2, Wfc, bfc):
    del tree_lens
    msk3 = attention_msk.reshape(_B, 1, _T * _L)
    wf2 = Wf.reshape(2, _H)
    bf2 = bf.reshape(1, 1)
    b1r = b1.reshape(1, _H)
    b2r = b2.reshape(1, _H)
    bfcr = bfc.reshape(1, _H)

    return pl.pallas_call(
        _gcn_pool_kernel,
        grid=(_B,),
        in_specs=[
            pl.BlockSpec((1, _T * _L, _H), lambda i: (i, 0, 0)),
            pl.BlockSpec((1, 1, _T * _L), lambda i: (i, 0, 0)),
            pl.BlockSpec((1, 2, _E), lambda i: (i, 0, 0)),
            pl.BlockSpec((2, _H), lambda i: (0, 0)),
            pl.BlockSpec((1, 1), lambda i: (0, 0)),
            pl.BlockSpec((_H, _H), lambda i: (0, 0)),
            pl.BlockSpec((1, _H), lambda i: (0, 0)),
            pl.BlockSpec((_H, _H), lambda i: (0, 0)),
            pl.BlockSpec((1, _H), lambda i: (0, 0)),
            pl.BlockSpec((_H, _H), lambda i: (0, 0)),
            pl.BlockSpec((1, _H), lambda i: (0, 0)),
        ],
        out_specs=pl.BlockSpec((1, 1, _H), lambda i: (i, 0, 0)),
        out_shape=jax.ShapeDtypeStruct((_B, 1, _H), jnp.float32),
        compiler_params=pltpu.CompilerParams(
            dimension_semantics=("parallel",)),
    )(hidden_states, msk3, edge_index, wf2, bf2,
      W1, b1r, W2, b2r, Wfc, bfcr).reshape(_B, _H)


# speedup vs baseline: 1.4719x; 1.0047x over previous
import jax
import jax.numpy as jnp
from jax.experimental import pallas as pl
from jax.experimental.pallas import tpu as pltpu

_B = 16
_T = 64
_L = 64
_H = 768
_E = 63
_HALF = 2
_R = _T * _L // _HALF
_TH = _T // _HALF
_NBUF = 4
_NCHUNK = _B * _HALF


def _gcn_pool_kernel(hs_ref, msk_ref, edge_ref, wf_ref, bf_ref,
                     w1_ref, b1_ref, w2_ref, b2_ref, wfc_ref, bfc_ref,
                     out_ref, buf_ref, nodes_ref, sem_ref):
    f32 = jnp.float32
    j = pl.program_id(0)
    h = j % _HALF

    def start(k):
        slot = jax.lax.rem(k, _NBUF)
        tree = k // _HALF
        half = jax.lax.rem(k, _HALF)
        pltpu.make_async_copy(
            hs_ref.at[tree, pl.ds(half * _R, _R), :],
            buf_ref.at[slot], sem_ref.at[slot]).start()

    @pl.when(j == 0)
    def _prime():
        for k in range(_NBUF - 1):
            start(k)

    @pl.when(j + _NBUF - 1 < _NCHUNK)
    def _next():
        start(j + _NBUF - 1)

    slot = jax.lax.rem(j, _NBUF)
    pltpu.make_async_copy(
        hs_ref.at[0, pl.ds(0, _R), :], buf_ref.at[slot],
        sem_ref.at[slot]).wait()
    hs = buf_ref[slot]

    m = msk_ref[0]
    col_tweet = jax.lax.broadcasted_iota(jnp.int32, (_TH, _R), 1) // _L
    row_tweet = jax.lax.broadcasted_iota(jnp.int32, (_TH, _R), 0)
    pool = jnp.where(col_tweet == row_tweet,
                     jnp.broadcast_to(m, (_TH, _R)), 0.0)
    sums = jnp.dot(pool, hs, preferred_element_type=f32)
    cnts = jnp.sum(pool, axis=1, keepdims=True)
    nodes_ref[pl.ds(h * _TH, _TH), :] = sums / jnp.maximum(cnts, 1e-9)

    @pl.when(h == _HALF - 1)
    def _graph():
        nodes = nodes_ref[...]

        src = edge_ref[0, 0:1, :]
        dst = edge_ref[0, 1:2, :]
        node_ids = jax.lax.broadcasted_iota(jnp.int32, (_T, _E), 0)
        oh_srcT = (node_ids == src).astype(f32)
        oh_dstT = (node_ids == dst).astype(f32)
        s_nodes = jax.lax.dot_general(wf_ref[...], nodes,
                                      (((1,), (1,)), ((), ())),
                                      preferred_element_type=f32)
        sc_e = jnp.dot(s_nodes[0:1, :], oh_dstT, preferred_element_type=f32)
        sp_e = jnp.dot(s_nodes[1:2, :], oh_srcT, preferred_element_type=f32)
        ew = jax.nn.sigmoid(sc_e + sp_e + bf_ref[0, 0])

        a_w = jax.lax.dot_general(oh_dstT * ew, oh_srcT,
                                  (((1,), (1,)), ((), ())),
                                  preferred_element_type=f32)
        eye = (jax.lax.broadcasted_iota(jnp.int32, (_T, _T), 0)
               == jax.lax.broadcasted_iota(jnp.int32, (_T, _T), 1)).astype(f32)
        a_sl = a_w + eye
        deg = jnp.sum(a_sl, axis=1, keepdims=True)
        dinv = jnp.where(deg > 0,
                         jax.lax.rsqrt(jnp.maximum(deg, 1e-12)), 0.0)

        h1 = jnp.dot(nodes, w1_ref[...], preferred_element_type=f32)
        x1 = jnp.maximum(dinv * jnp.dot(a_sl, dinv * h1,
                                        preferred_element_type=f32)
                         + b1_ref[...], 0.0)
        h2 = jnp.dot(x1, w2_ref[...], preferred_element_type=f32)
        x2 = jnp.maximum(dinv * jnp.dot(a_sl, dinv * h2,
                                        preferred_element_type=f32)
                         + b2_ref[...], 0.0)

        pooled = jnp.sum(x2, axis=0, keepdims=True) * (1.0 / _T)
        out_ref[0] = jnp.tanh(jnp.dot(pooled, wfc_ref[...],
                                      preferred_element_type=f32)
                              + bfc_ref[...])


def kernel(hidden_states, attention_msk, tree_lens, edge_index,
           Wf, bf, W1, b1, W2, b2, Wfc, bfc):
    del tree_lens
    msk3 = attention_msk.reshape(_NCHUNK, 1, _R)
    wf2 = Wf.reshape(2, _H)
    bf2 = bf.reshape(1, 1)
    b1r = b1.reshape(1, _H)
    b2r = b2.reshape(1, _H)
    bfcr = bfc.reshape(1, _H)

    return pl.pallas_call(
        _gcn_pool_kernel,
        grid=(_NCHUNK,),
        in_specs=[
            pl.BlockSpec(memory_space=pl.ANY),
            pl.BlockSpec((1, 1, _R), lambda j: (j, 0, 0)),
            pl.BlockSpec((1, 2, _E), lambda j: (j // _HALF, 0, 0)),
            pl.BlockSpec((2, _H), lambda j: (0, 0)),
            pl.BlockSpec((1, 1), lambda j: (0, 0)),
            pl.BlockSpec((_H, _H), lambda j: (0, 0)),
            pl.BlockSpec((1, _H), lambda j: (0, 0)),
            pl.BlockSpec((_H, _H), lambda j: (0, 0)),
            pl.BlockSpec((1, _H), lambda j: (0, 0)),
            pl.BlockSpec((_H, _H), lambda j: (0, 0)),
            pl.BlockSpec((1, _H), lambda j: (0, 0)),
        ],
        out_specs=pl.BlockSpec((1, 1, _H), lambda j: (j // _HALF, 0, 0)),
        out_shape=jax.ShapeDtypeStruct((_B, 1, _H), jnp.float32),
        scratch_shapes=[
            pltpu.VMEM((_NBUF, _R, _H), jnp.float32),
            pltpu.VMEM((_T, _H), jnp.float32),
            pltpu.SemaphoreType.DMA((_NBUF,)),
        ],
        compiler_params=pltpu.CompilerParams(
            dimension_semantics=("arbitrary",)),
    )(hidden_states, msk3, edge_index, wf2, bf2,
      W1, b1r, W2, b2r, Wfc, bfcr).reshape(_B, _H)
